# baseline (speedup 1.0000x reference)
import functools

import jax
import jax.numpy as jnp
from jax import lax
from jax.experimental import pallas as pl
from jax.experimental.pallas import tpu as pltpu
from jax.experimental.pallas import tpu_sc as plsc

DIM = 128
N = 10000
N_VAR = 6000
N_CON = 4000
E = 320000

NP_ = 10240
ZROW = N
EPT = NP_
NCHUNK = 80
GW = 128
NW = 32
STRIPE = NP_ // 16

AVP = 6144


def _mesh():
    return plsc.VectorSubcoreMesh(core_axis_name="c", subcore_axis_name="s")


def _spmm(h, src_p, dst_p):
    @functools.partial(
        pl.kernel,
        out_type=jax.ShapeDtypeStruct((2, NP_, DIM), jnp.float32),
        mesh=_mesh(),
        scratch_types=[
            pltpu.VMEM((NCHUNK, GW), jnp.int32),
            pltpu.VMEM((NCHUNK, GW), jnp.int32),
            pltpu.VMEM((GW, DIM), jnp.float32),
            pltpu.VMEM_SHARED((NP_, DIM), jnp.float32),
        ],
    )
    def k(h_hbm, src_hbm, dst_hbm, out_hbm, src_v, dst_v, buf, acc):
        cid = lax.axis_index("c")
        sid = lax.axis_index("s")
        wid = cid * 16 + sid
        pltpu.sync_copy(src_hbm.at[wid], src_v)
        pltpu.sync_copy(dst_hbm.at[wid], dst_v)

        zv = jnp.zeros((16,), jnp.float32)

        def zrow(r, _):
            for j in range(DIM // 16):
                buf[r, pl.ds(j * 16, 16)] = zv
            return 0

        lax.fori_loop(0, GW, zrow, 0)
        row0 = sid * STRIPE
        for j in range(STRIPE // GW):
            pltpu.sync_copy(buf, acc.at[pl.ds(row0 + j * GW, GW)])
        plsc.subcore_barrier()

        @pl.when(wid == 0)
        def _serial():
            def per_tile(w, _):
                pltpu.sync_copy(src_hbm.at[w], src_v)
                pltpu.sync_copy(dst_hbm.at[w], dst_v)

                def chunk(c, _):
                    pltpu.sync_copy(h_hbm.at[src_v.at[c]], buf)
                    pltpu.sync_copy(buf, acc.at[dst_v.at[c]], add=True)
                    return 0

                lax.fori_loop(0, NCHUNK, chunk, 0)
                return 0

            lax.fori_loop(0, NW, per_tile, 0)

        plsc.subcore_barrier()
        pltpu.sync_copy(acc.at[pl.ds(row0, STRIPE)],
                        out_hbm.at[cid, pl.ds(row0, STRIPE)])

    return k(h, src_p, dst_p)


def _gather_rows(table, sel_p, rows_per_chunk, chunks):
    rpt = rows_per_chunk * chunks

    @functools.partial(
        pl.kernel,
        out_type=jax.ShapeDtypeStruct((NW * rpt, DIM), jnp.float32),
        mesh=_mesh(),
        scratch_types=[
            pltpu.VMEM((chunks, rows_per_chunk), jnp.int32),
            pltpu.VMEM((rows_per_chunk, DIM), jnp.float32),
        ],
    )
    def k(tbl_hbm, sel_hbm, out_hbm, idx_v, buf):
        cid = lax.axis_index("c")
        sid = lax.axis_index("s")
        wid = cid * 16 + sid
        pltpu.sync_copy(sel_hbm.at[wid], idx_v)
        for c in range(chunks):
            pltpu.sync_copy(tbl_hbm.at[idx_v.at[c]], buf)
            pltpu.sync_copy(buf, out_hbm.at[pl.ds(wid * rpt + c * rows_per_chunk,
                                                  rows_per_chunk)])

    return k(table, sel_p)


def _gather5(hs, av_p):
    @functools.partial(
        pl.kernel,
        out_type=jax.ShapeDtypeStruct((5, AVP, DIM), jnp.float32),
        mesh=_mesh(),
        scratch_types=[
            pltpu.VMEM((2, 96), jnp.int32),
            pltpu.VMEM((96, DIM), jnp.float32),
        ],
    )
    def k(h0, h1, h2, h3, h4, av_hbm, out_hbm, idx_v, buf):
        cid = lax.axis_index("c")
        sid = lax.axis_index("s")
        wid = cid * 16 + sid
        pltpu.sync_copy(av_hbm.at[wid], idx_v)
        for t, hh in enumerate((h0, h1, h2, h3, h4)):
            for c in range(2):
                pltpu.sync_copy(hh.at[idx_v.at[c]], buf)
                pltpu.sync_copy(buf, out_hbm.at[t, pl.ds(wid * 192 + c * 96, 96)])

    return k(*hs, av_p)


def _embed(v2, w1row, b1, w2, b2):
    rows = v2.shape[0]
    blk = 1000

    def body(v_ref, w1_ref, b1_ref, w2_ref, b2_ref, o_ref):
        v = v_ref[...]
        h1 = jax.nn.relu(v[:, 0:1] * w1_ref[...] + b1_ref[...])
        h2 = jnp.dot(h1, w2_ref[...], preferred_element_type=jnp.float32) + b2_ref[...]
        mask = lax.broadcasted_iota(jnp.int32, (1, DIM), 1) == (DIM - 1)
        o_ref[...] = jnp.where(mask, v, h2)

    return pl.pallas_call(
        body,
        grid=(rows // blk,),
        in_specs=[
            pl.BlockSpec((blk, DIM), lambda i: (i, 0)),
            pl.BlockSpec((1, DIM), lambda i: (0, 0)),
            pl.BlockSpec((1, DIM), lambda i: (0, 0)),
            pl.BlockSpec((DIM, DIM), lambda i: (0, 0)),
            pl.BlockSpec((1, DIM), lambda i: (0, 0)),
        ],
        out_specs=pl.BlockSpec((blk, DIM), lambda i: (i, 0)),
        out_shape=jax.ShapeDtypeStruct((rows, DIM), jnp.float32),
    )(v2, w1row, b1, w2, b2)


def _msg_proj(h, wcons):
    blk = 1280

    def body(h_ref, w_ref, o_ref):
        o_ref[...] = jnp.dot(h_ref[...], w_ref[...],
                             preferred_element_type=jnp.float32)

    return pl.pallas_call(
        body,
        grid=(NP_ // blk,),
        in_specs=[
            pl.BlockSpec((blk, DIM), lambda i: (i, 0)),
            pl.BlockSpec((DIM, DIM), lambda i: (0, 0)),
        ],
        out_specs=pl.BlockSpec((blk, DIM), lambda i: (i, 0)),
        out_shape=jax.ShapeDtypeStruct((NP_, DIM), jnp.float32),
    )(h, wcons)


def _conv_update(part, h, root, bias2d):
    blk = 1280

    def body(p_ref, h_ref, r_ref, b_ref, o_ref):
        a = p_ref[0] + p_ref[1]
        o_ref[...] = jax.nn.relu(
            a
            + jnp.dot(h_ref[...], r_ref[...], preferred_element_type=jnp.float32)
            + b_ref[...])

    return pl.pallas_call(
        body,
        grid=(NP_ // blk,),
        in_specs=[
            pl.BlockSpec((2, blk, DIM), lambda i: (0, i, 0)),
            pl.BlockSpec((blk, DIM), lambda i: (i, 0)),
            pl.BlockSpec((DIM, DIM), lambda i: (0, 0)),
            pl.BlockSpec((1, DIM), lambda i: (0, 0)),
        ],
        out_specs=pl.BlockSpec((blk, DIM), lambda i: (i, 0)),
        out_shape=jax.ShapeDtypeStruct((NP_, DIM), jnp.float32),
    )(part, h, root, bias2d)


def _fc_head(g, w1r, b1, w2, b2, w3, b3, w4p, b4p):
    blk = 768

    def body(g_ref, w1_ref, b1_ref, w2_ref, b2_ref, w3_ref, b3_ref,
             w4_ref, b4_ref, o_ref):
        acc = b1_ref[...] + jnp.zeros((blk, DIM), jnp.float32)
        for t in range(5):
            acc = acc + jnp.dot(g_ref[t], w1_ref[t],
                                preferred_element_type=jnp.float32)
        hh = jax.nn.relu(acc)
        hh = jax.nn.relu(jnp.dot(hh, w2_ref[...],
                                 preferred_element_type=jnp.float32) + b2_ref[...])
        hh = jax.nn.relu(jnp.dot(hh, w3_ref[...],
                                 preferred_element_type=jnp.float32) + b3_ref[...])
        o_ref[...] = jnp.dot(hh, w4_ref[...],
                             preferred_element_type=jnp.float32) + b4_ref[...]

    return pl.pallas_call(
        body,
        grid=(AVP // blk,),
        in_specs=[
            pl.BlockSpec((5, blk, DIM), lambda i: (0, i, 0)),
            pl.BlockSpec((5, DIM, DIM), lambda i: (0, 0, 0)),
            pl.BlockSpec((1, DIM), lambda i: (0, 0)),
            pl.BlockSpec((DIM, DIM), lambda i: (0, 0)),
            pl.BlockSpec((1, DIM), lambda i: (0, 0)),
            pl.BlockSpec((DIM, DIM), lambda i: (0, 0)),
            pl.BlockSpec((1, DIM), lambda i: (0, 0)),
            pl.BlockSpec((DIM, DIM), lambda i: (0, 0)),
            pl.BlockSpec((1, DIM), lambda i: (0, 0)),
        ],
        out_specs=pl.BlockSpec((blk, DIM), lambda i: (i, 0)),
        out_shape=jax.ShapeDtypeStruct((AVP, DIM), jnp.float32),
    )(g, w1r, b1, w2, b2, w3, b3, w4p, b4p)


@jax.jit
def kernel(var_node_features, con_node_features, node_types, assoc_var, assoc_con,
           edge_index, edge_types, edge_features,
           vm_w1, vm_b1, vm_w2, vm_b2, cm_w1, cm_b1, cm_w2, cm_b2,
           conv1_wcons, conv1_root, conv1_bias,
           conv2_wcons, conv2_root, conv2_bias,
           conv3_wcons, conv3_root, conv3_bias,
           conv4_wcons, conv4_root, conv4_bias,
           fc1_w, fc1_b, fc2_w, fc2_b, fc3_w, fc3_b, fc4_w, fc4_b):
    f32 = jnp.float32

    io_mask = ((lax.broadcasted_iota(jnp.int32, (1, DIM), 1) == 0) |
               (lax.broadcasted_iota(jnp.int32, (1, DIM), 1) == DIM - 1)
               ).astype(f32)
    v2 = var_node_features * io_mask
    c2 = con_node_features * io_mask

    def pad_mlp_weights(w1, b1, w2, b2):
        w1p = jnp.pad(w1, ((0, 0), (0, 1)))
        b1p = jnp.pad(b1, (0, 1)).reshape(1, DIM)
        w2p = jnp.zeros((DIM, DIM), f32).at[: DIM - 1, : DIM - 1].set(w2)
        b2p = jnp.pad(b2, (0, 1)).reshape(1, DIM)
        return w1p, b1p, w2p, b2p

    n_emb = _embed(v2, *pad_mlp_weights(vm_w1, vm_b1, vm_w2, vm_b2))
    e_emb = _embed(c2, *pad_mlp_weights(cm_w1, cm_b1, cm_w2, cm_b2))
    table = jnp.concatenate([n_emb, e_emb, jnp.zeros((NP_ - N, DIM), f32)], axis=0)

    sel = jnp.full((N,), ZROW, jnp.int32)
    sel = sel.at[assoc_var].set(jnp.arange(N_VAR, dtype=jnp.int32))
    sel = sel.at[assoc_con].set(N_VAR + jnp.arange(N_CON, dtype=jnp.int32))
    sel_p = jnp.pad(sel, (0, NP_ - N), constant_values=ZROW).reshape(NW, 4, 80)
    x = _gather_rows(table, sel_p, 80, 4)

    src_p = jnp.pad(edge_index[0], (0, NW * EPT - E)).reshape(NW, NCHUNK, GW)
    dst_p = jnp.pad(edge_index[1], (0, NW * EPT - E),
                    constant_values=ZROW).reshape(NW, NCHUNK, GW)

    convs = ((conv1_wcons, conv1_root, conv1_bias),
             (conv2_wcons, conv2_root, conv2_bias),
             (conv3_wcons, conv3_root, conv3_bias),
             (conv4_wcons, conv4_root, conv4_bias))
    hs = [x]
    h = x
    for li, (wcons, root, bias) in enumerate(convs):
        proj = _msg_proj(h, wcons)
        part = _spmm(proj, src_p, dst_p)
        agg = part[0] + part[1]
        h = jax.nn.relu(agg + h @ root + bias)
        hs.append(h)

    av_p = jnp.pad(assoc_var, (0, AVP - N_VAR),
                   constant_values=ZROW).reshape(NW, 2, 96)
    g = _gather5(hs, av_p)
    w4p = jnp.pad(fc4_w, ((0, 0), (0, DIM - 1)))
    b4p = jnp.pad(fc4_b, (0, DIM - 1)).reshape(1, DIM)
    out = _fc_head(g, fc1_w.reshape(5, DIM, DIM), fc1_b.reshape(1, DIM),
                   fc2_w, fc2_b.reshape(1, DIM), fc3_w, fc3_b.reshape(1, DIM),
                   w4p, b4p)
    return out[:N_VAR, 0]


# speedup vs baseline: 7.3535x; 7.3535x over previous
import functools

import jax
import jax.numpy as jnp
from jax import lax
from jax.experimental import pallas as pl
from jax.experimental.pallas import tpu as pltpu
from jax.experimental.pallas import tpu_sc as plsc

DIM = 128
N = 10000
N_VAR = 6000
N_CON = 4000
E = 320000

NP_ = 10240
ZROW = N
EPT = NP_
NCHUNK = 80
GW = 128
NW = 32
STRIPE = NP_ // 16

AVP = 6144


def _mesh():
    return plsc.VectorSubcoreMesh(core_axis_name="c", subcore_axis_name="s")


def _spmm(h, src_p, dst_p):
    @functools.partial(
        pl.kernel,
        out_type=jax.ShapeDtypeStruct((2, NP_, DIM), jnp.float32),
        mesh=_mesh(),
        scratch_types=[
            pltpu.VMEM((NCHUNK, GW), jnp.int32),
            pltpu.VMEM((NCHUNK, GW), jnp.int32),
            pltpu.VMEM((GW, DIM), jnp.float32),
            pltpu.VMEM_SHARED((NP_, DIM), jnp.float32),
        ],
    )
    def k(h_hbm, src_hbm, dst_hbm, out_hbm, src_v, dst_v, buf, acc):
        cid = lax.axis_index("c")
        sid = lax.axis_index("s")
        wid = cid * 16 + sid
        pltpu.sync_copy(src_hbm.at[wid], src_v)
        pltpu.sync_copy(dst_hbm.at[wid], dst_v)

        zv = jnp.zeros((16,), jnp.float32)

        def zrow(r, _):
            for j in range(DIM // 16):
                buf[r, pl.ds(j * 16, 16)] = zv
            return 0

        lax.fori_loop(0, GW, zrow, 0)
        row0 = sid * STRIPE
        for j in range(STRIPE // GW):
            pltpu.sync_copy(buf, acc.at[pl.ds(row0 + j * GW, GW)])
        plsc.subcore_barrier()

        def chunk(c, _):
            pltpu.sync_copy(h_hbm.at[src_v.at[c]], buf)
            pltpu.sync_copy(buf, acc.at[dst_v.at[c]], add=True)
            return 0

        lax.fori_loop(0, NCHUNK, chunk, 0)
        plsc.subcore_barrier()
        pltpu.sync_copy(acc.at[pl.ds(row0, STRIPE)],
                        out_hbm.at[cid, pl.ds(row0, STRIPE)])

    return k(h, src_p, dst_p)


def _gather_rows(table, sel_p, rows_per_chunk, chunks):
    rpt = rows_per_chunk * chunks

    @functools.partial(
        pl.kernel,
        out_type=jax.ShapeDtypeStruct((NW * rpt, DIM), jnp.float32),
        mesh=_mesh(),
        scratch_types=[
            pltpu.VMEM((chunks, rows_per_chunk), jnp.int32),
            pltpu.VMEM((rows_per_chunk, DIM), jnp.float32),
        ],
    )
    def k(tbl_hbm, sel_hbm, out_hbm, idx_v, buf):
        cid = lax.axis_index("c")
        sid = lax.axis_index("s")
        wid = cid * 16 + sid
        pltpu.sync_copy(sel_hbm.at[wid], idx_v)
        for c in range(chunks):
            pltpu.sync_copy(tbl_hbm.at[idx_v.at[c]], buf)
            pltpu.sync_copy(buf, out_hbm.at[pl.ds(wid * rpt + c * rows_per_chunk,
                                                  rows_per_chunk)])

    return k(table, sel_p)


def _gather5(hs, av_p):
    @functools.partial(
        pl.kernel,
        out_type=jax.ShapeDtypeStruct((5, AVP, DIM), jnp.float32),
        mesh=_mesh(),
        scratch_types=[
            pltpu.VMEM((2, 96), jnp.int32),
            pltpu.VMEM((96, DIM), jnp.float32),
        ],
    )
    def k(h0, h1, h2, h3, h4, av_hbm, out_hbm, idx_v, buf):
        cid = lax.axis_index("c")
        sid = lax.axis_index("s")
        wid = cid * 16 + sid
        pltpu.sync_copy(av_hbm.at[wid], idx_v)
        for t, hh in enumerate((h0, h1, h2, h3, h4)):
            for c in range(2):
                pltpu.sync_copy(hh.at[idx_v.at[c]], buf)
                pltpu.sync_copy(buf, out_hbm.at[t, pl.ds(wid * 192 + c * 96, 96)])

    return k(*hs, av_p)


def _embed(v2, w1row, b1, w2, b2):
    rows = v2.shape[0]
    blk = 1000

    def body(v_ref, w1_ref, b1_ref, w2_ref, b2_ref, o_ref):
        v = v_ref[...]
        h1 = jax.nn.relu(v[:, 0:1] * w1_ref[...] + b1_ref[...])
        h2 = jnp.dot(h1, w2_ref[...], preferred_element_type=jnp.float32) + b2_ref[...]
        mask = lax.broadcasted_iota(jnp.int32, (1, DIM), 1) == (DIM - 1)
        o_ref[...] = jnp.where(mask, v, h2)

    return pl.pallas_call(
        body,
        grid=(rows // blk,),
        in_specs=[
            pl.BlockSpec((blk, DIM), lambda i: (i, 0)),
            pl.BlockSpec((1, DIM), lambda i: (0, 0)),
            pl.BlockSpec((1, DIM), lambda i: (0, 0)),
            pl.BlockSpec((DIM, DIM), lambda i: (0, 0)),
            pl.BlockSpec((1, DIM), lambda i: (0, 0)),
        ],
        out_specs=pl.BlockSpec((blk, DIM), lambda i: (i, 0)),
        out_shape=jax.ShapeDtypeStruct((rows, DIM), jnp.float32),
    )(v2, w1row, b1, w2, b2)


def _msg_proj(h, wcons):
    blk = 1280

    def body(h_ref, w_ref, o_ref):
        o_ref[...] = jnp.dot(h_ref[...], w_ref[...],
                             preferred_element_type=jnp.float32)

    return pl.pallas_call(
        body,
        grid=(NP_ // blk,),
        in_specs=[
            pl.BlockSpec((blk, DIM), lambda i: (i, 0)),
            pl.BlockSpec((DIM, DIM), lambda i: (0, 0)),
        ],
        out_specs=pl.BlockSpec((blk, DIM), lambda i: (i, 0)),
        out_shape=jax.ShapeDtypeStruct((NP_, DIM), jnp.float32),
    )(h, wcons)


def _conv_update(part, h, root, bias2d):
    blk = 1280

    def body(p_ref, h_ref, r_ref, b_ref, o_ref):
        a = p_ref[0] + p_ref[1]
        o_ref[...] = jax.nn.relu(
            a
            + jnp.dot(h_ref[...], r_ref[...], preferred_element_type=jnp.float32)
            + b_ref[...])

    return pl.pallas_call(
        body,
        grid=(NP_ // blk,),
        in_specs=[
            pl.BlockSpec((2, blk, DIM), lambda i: (0, i, 0)),
            pl.BlockSpec((blk, DIM), lambda i: (i, 0)),
            pl.BlockSpec((DIM, DIM), lambda i: (0, 0)),
            pl.BlockSpec((1, DIM), lambda i: (0, 0)),
        ],
        out_specs=pl.BlockSpec((blk, DIM), lambda i: (i, 0)),
        out_shape=jax.ShapeDtypeStruct((NP_, DIM), jnp.float32),
    )(part, h, root, bias2d)


def _fc_head(g, w1r, b1, w2, b2, w3, b3, w4p, b4p):
    blk = 768

    def body(g_ref, w1_ref, b1_ref, w2_ref, b2_ref, w3_ref, b3_ref,
             w4_ref, b4_ref, o_ref):
        acc = b1_ref[...] + jnp.zeros((blk, DIM), jnp.float32)
        for t in range(5):
            acc = acc + jnp.dot(g_ref[t], w1_ref[t],
                                preferred_element_type=jnp.float32)
        hh = jax.nn.relu(acc)
        hh = jax.nn.relu(jnp.dot(hh, w2_ref[...],
                                 preferred_element_type=jnp.float32) + b2_ref[...])
        hh = jax.nn.relu(jnp.dot(hh, w3_ref[...],
                                 preferred_element_type=jnp.float32) + b3_ref[...])
        o_ref[...] = jnp.dot(hh, w4_ref[...],
                             preferred_element_type=jnp.float32) + b4_ref[...]

    return pl.pallas_call(
        body,
        grid=(AVP // blk,),
        in_specs=[
            pl.BlockSpec((5, blk, DIM), lambda i: (0, i, 0)),
            pl.BlockSpec((5, DIM, DIM), lambda i: (0, 0, 0)),
            pl.BlockSpec((1, DIM), lambda i: (0, 0)),
            pl.BlockSpec((DIM, DIM), lambda i: (0, 0)),
            pl.BlockSpec((1, DIM), lambda i: (0, 0)),
            pl.BlockSpec((DIM, DIM), lambda i: (0, 0)),
            pl.BlockSpec((1, DIM), lambda i: (0, 0)),
            pl.BlockSpec((DIM, DIM), lambda i: (0, 0)),
            pl.BlockSpec((1, DIM), lambda i: (0, 0)),
        ],
        out_specs=pl.BlockSpec((blk, DIM), lambda i: (i, 0)),
        out_shape=jax.ShapeDtypeStruct((AVP, DIM), jnp.float32),
    )(g, w1r, b1, w2, b2, w3, b3, w4p, b4p)


@jax.jit
def kernel(var_node_features, con_node_features, node_types, assoc_var, assoc_con,
           edge_index, edge_types, edge_features,
           vm_w1, vm_b1, vm_w2, vm_b2, cm_w1, cm_b1, cm_w2, cm_b2,
           conv1_wcons, conv1_root, conv1_bias,
           conv2_wcons, conv2_root, conv2_bias,
           conv3_wcons, conv3_root, conv3_bias,
           conv4_wcons, conv4_root, conv4_bias,
           fc1_w, fc1_b, fc2_w, fc2_b, fc3_w, fc3_b, fc4_w, fc4_b):
    f32 = jnp.float32

    io_mask = ((lax.broadcasted_iota(jnp.int32, (1, DIM), 1) == 0) |
               (lax.broadcasted_iota(jnp.int32, (1, DIM), 1) == DIM - 1)
               ).astype(f32)
    v2 = var_node_features * io_mask
    c2 = con_node_features * io_mask

    def pad_mlp_weights(w1, b1, w2, b2):
        w1p = jnp.pad(w1, ((0, 0), (0, 1)))
        b1p = jnp.pad(b1, (0, 1)).reshape(1, DIM)
        w2p = jnp.zeros((DIM, DIM), f32).at[: DIM - 1, : DIM - 1].set(w2)
        b2p = jnp.pad(b2, (0, 1)).reshape(1, DIM)
        return w1p, b1p, w2p, b2p

    n_emb = _embed(v2, *pad_mlp_weights(vm_w1, vm_b1, vm_w2, vm_b2))
    e_emb = _embed(c2, *pad_mlp_weights(cm_w1, cm_b1, cm_w2, cm_b2))
    table = jnp.concatenate([n_emb, e_emb, jnp.zeros((NP_ - N, DIM), f32)], axis=0)

    sel = jnp.full((N,), ZROW, jnp.int32)
    sel = sel.at[assoc_var].set(jnp.arange(N_VAR, dtype=jnp.int32))
    sel = sel.at[assoc_con].set(N_VAR + jnp.arange(N_CON, dtype=jnp.int32))
    sel_p = jnp.pad(sel, (0, NP_ - N), constant_values=ZROW).reshape(NW, 4, 80)
    x = _gather_rows(table, sel_p, 80, 4)

    perm = jnp.argsort(edge_index[1])
    src_s = jnp.take(edge_index[0], perm)
    dst_s = jnp.take(edge_index[1], perm)
    src_p = jnp.pad(src_s, (0, NW * EPT - E)).reshape(NW, NCHUNK, GW)
    dst_p = jnp.pad(dst_s, (0, NW * EPT - E),
                    constant_values=ZROW).reshape(NW, NCHUNK, GW)

    convs = ((conv1_wcons, conv1_root, conv1_bias),
             (conv2_wcons, conv2_root, conv2_bias),
             (conv3_wcons, conv3_root, conv3_bias),
             (conv4_wcons, conv4_root, conv4_bias))
    hs = [x]
    h = x
    for li, (wcons, root, bias) in enumerate(convs):
        proj = _msg_proj(h, wcons)
        part = _spmm(proj, src_p, dst_p)
        agg = part[0] + part[1]
        h = jax.nn.relu(agg + h @ root + bias)
        hs.append(h)

    av_p = jnp.pad(assoc_var, (0, AVP - N_VAR),
                   constant_values=ZROW).reshape(NW, 2, 96)
    g = _gather5(hs, av_p)
    w4p = jnp.pad(fc4_w, ((0, 0), (0, DIM - 1)))
    b4p = jnp.pad(fc4_b, (0, DIM - 1)).reshape(1, DIM)
    out = _fc_head(g, fc1_w.reshape(5, DIM, DIM), fc1_b.reshape(1, DIM),
                   fc2_w, fc2_b.reshape(1, DIM), fc3_w, fc3_b.reshape(1, DIM),
                   w4p, b4p)
    return out[:N_VAR, 0]


# speedup vs baseline: 14.5494x; 1.9786x over previous
import functools

import jax
import jax.numpy as jnp
from jax import lax
from jax.experimental import pallas as pl
from jax.experimental.pallas import tpu as pltpu
from jax.experimental.pallas import tpu_sc as plsc

DIM = 128
N = 10000
N_VAR = 6000
N_CON = 4000
E = 320000

NP_ = 10240
ZROW = N
EPT = NP_
NCHUNK = 80
GW = 128
NW = 32
STRIPE = NP_ // 16

AVP = 6144


def _mesh():
    return plsc.VectorSubcoreMesh(core_axis_name="c", subcore_axis_name="s")


def _spmm(h, src_p, dst_p):
    @functools.partial(
        pl.kernel,
        out_type=jax.ShapeDtypeStruct((2, NP_, DIM), jnp.float32),
        mesh=_mesh(),
        scratch_types=[
            pltpu.VMEM((NCHUNK, GW), jnp.int32),
            pltpu.VMEM((NCHUNK, GW), jnp.int32),
            pltpu.VMEM((GW, DIM), jnp.float32),
            pltpu.VMEM_SHARED((NP_, DIM), jnp.float32),
        ],
    )
    def k(h_hbm, src_hbm, dst_hbm, out_hbm, src_v, dst_v, buf, acc):
        cid = lax.axis_index("c")
        sid = lax.axis_index("s")
        wid = cid * 16 + sid
        pltpu.sync_copy(src_hbm.at[wid], src_v)
        pltpu.sync_copy(dst_hbm.at[wid], dst_v)

        zv = jnp.zeros((16,), jnp.float32)

        def zrow(r, _):
            for j in range(DIM // 16):
                buf[r, pl.ds(j * 16, 16)] = zv
            return 0

        lax.fori_loop(0, GW, zrow, 0)
        row0 = sid * STRIPE
        for j in range(STRIPE // GW):
            pltpu.sync_copy(buf, acc.at[pl.ds(row0 + j * GW, GW)])
        plsc.subcore_barrier()

        def chunk(c, _):
            pltpu.sync_copy(h_hbm.at[src_v.at[c]], buf)
            pltpu.sync_copy(buf, acc.at[dst_v.at[c]], add=True)
            return 0

        lax.fori_loop(0, NCHUNK, chunk, 0)
        plsc.subcore_barrier()
        pltpu.sync_copy(acc.at[pl.ds(row0, STRIPE)],
                        out_hbm.at[cid, pl.ds(row0, STRIPE)])

    return k(h, src_p, dst_p)


def _gather_rows(table, sel_p, rows_per_chunk, chunks):
    rpt = rows_per_chunk * chunks

    @functools.partial(
        pl.kernel,
        out_type=jax.ShapeDtypeStruct((NW * rpt, DIM), jnp.float32),
        mesh=_mesh(),
        scratch_types=[
            pltpu.VMEM((chunks, rows_per_chunk), jnp.int32),
            pltpu.VMEM((rows_per_chunk, DIM), jnp.float32),
        ],
    )
    def k(tbl_hbm, sel_hbm, out_hbm, idx_v, buf):
        cid = lax.axis_index("c")
        sid = lax.axis_index("s")
        wid = cid * 16 + sid
        pltpu.sync_copy(sel_hbm.at[wid], idx_v)
        for c in range(chunks):
            pltpu.sync_copy(tbl_hbm.at[idx_v.at[c]], buf)
            pltpu.sync_copy(buf, out_hbm.at[pl.ds(wid * rpt + c * rows_per_chunk,
                                                  rows_per_chunk)])

    return k(table, sel_p)


def _gather5(hs, av_p):
    @functools.partial(
        pl.kernel,
        out_type=jax.ShapeDtypeStruct((5, AVP, DIM), jnp.float32),
        mesh=_mesh(),
        scratch_types=[
            pltpu.VMEM((2, 96), jnp.int32),
            pltpu.VMEM((96, DIM), jnp.float32),
        ],
    )
    def k(h0, h1, h2, h3, h4, av_hbm, out_hbm, idx_v, buf):
        cid = lax.axis_index("c")
        sid = lax.axis_index("s")
        wid = cid * 16 + sid
        pltpu.sync_copy(av_hbm.at[wid], idx_v)
        for t, hh in enumerate((h0, h1, h2, h3, h4)):
            for c in range(2):
                pltpu.sync_copy(hh.at[idx_v.at[c]], buf)
                pltpu.sync_copy(buf, out_hbm.at[t, pl.ds(wid * 192 + c * 96, 96)])

    return k(*hs, av_p)


def _embed(v2, w1row, b1, w2, b2):
    rows = v2.shape[0]
    blk = 1000

    def body(v_ref, w1_ref, b1_ref, w2_ref, b2_ref, o_ref):
        v = v_ref[...]
        h1 = jax.nn.relu(v[:, 0:1] * w1_ref[...] + b1_ref[...])
        h2 = jnp.dot(h1, w2_ref[...], preferred_element_type=jnp.float32) + b2_ref[...]
        mask = lax.broadcasted_iota(jnp.int32, (1, DIM), 1) == (DIM - 1)
        o_ref[...] = jnp.where(mask, v, h2)

    return pl.pallas_call(
        body,
        grid=(rows // blk,),
        in_specs=[
            pl.BlockSpec((blk, DIM), lambda i: (i, 0)),
            pl.BlockSpec((1, DIM), lambda i: (0, 0)),
            pl.BlockSpec((1, DIM), lambda i: (0, 0)),
            pl.BlockSpec((DIM, DIM), lambda i: (0, 0)),
            pl.BlockSpec((1, DIM), lambda i: (0, 0)),
        ],
        out_specs=pl.BlockSpec((blk, DIM), lambda i: (i, 0)),
        out_shape=jax.ShapeDtypeStruct((rows, DIM), jnp.float32),
    )(v2, w1row, b1, w2, b2)


def _msg_proj(h, wcons):
    blk = 1280

    def body(h_ref, w_ref, o_ref):
        o_ref[...] = jnp.dot(h_ref[...], w_ref[...],
                             preferred_element_type=jnp.float32)

    return pl.pallas_call(
        body,
        grid=(NP_ // blk,),
        in_specs=[
            pl.BlockSpec((blk, DIM), lambda i: (i, 0)),
            pl.BlockSpec((DIM, DIM), lambda i: (0, 0)),
        ],
        out_specs=pl.BlockSpec((blk, DIM), lambda i: (i, 0)),
        out_shape=jax.ShapeDtypeStruct((NP_, DIM), jnp.float32),
    )(h, wcons)


def _conv_update(part, h, root, bias2d):
    blk = 1280

    def body(p_ref, h_ref, r_ref, b_ref, o_ref):
        a = p_ref[0] + p_ref[1]
        o_ref[...] = jax.nn.relu(
            a
            + jnp.dot(h_ref[...], r_ref[...], preferred_element_type=jnp.float32)
            + b_ref[...])

    return pl.pallas_call(
        body,
        grid=(NP_ // blk,),
        in_specs=[
            pl.BlockSpec((2, blk, DIM), lambda i: (0, i, 0)),
            pl.BlockSpec((blk, DIM), lambda i: (i, 0)),
            pl.BlockSpec((DIM, DIM), lambda i: (0, 0)),
            pl.BlockSpec((1, DIM), lambda i: (0, 0)),
        ],
        out_specs=pl.BlockSpec((blk, DIM), lambda i: (i, 0)),
        out_shape=jax.ShapeDtypeStruct((NP_, DIM), jnp.float32),
    )(part, h, root, bias2d)


def _fc_head(g, w1r, b1, w2, b2, w3, b3, w4p, b4p):
    blk = 768

    def body(g_ref, w1_ref, b1_ref, w2_ref, b2_ref, w3_ref, b3_ref,
             w4_ref, b4_ref, o_ref):
        acc = b1_ref[...] + jnp.zeros((blk, DIM), jnp.float32)
        for t in range(5):
            acc = acc + jnp.dot(g_ref[t], w1_ref[t],
                                preferred_element_type=jnp.float32)
        hh = jax.nn.relu(acc)
        hh = jax.nn.relu(jnp.dot(hh, w2_ref[...],
                                 preferred_element_type=jnp.float32) + b2_ref[...])
        hh = jax.nn.relu(jnp.dot(hh, w3_ref[...],
                                 preferred_element_type=jnp.float32) + b3_ref[...])
        o_ref[...] = jnp.dot(hh, w4_ref[...],
                             preferred_element_type=jnp.float32) + b4_ref[...]

    return pl.pallas_call(
        body,
        grid=(AVP // blk,),
        in_specs=[
            pl.BlockSpec((5, blk, DIM), lambda i: (0, i, 0)),
            pl.BlockSpec((5, DIM, DIM), lambda i: (0, 0, 0)),
            pl.BlockSpec((1, DIM), lambda i: (0, 0)),
            pl.BlockSpec((DIM, DIM), lambda i: (0, 0)),
            pl.BlockSpec((1, DIM), lambda i: (0, 0)),
            pl.BlockSpec((DIM, DIM), lambda i: (0, 0)),
            pl.BlockSpec((1, DIM), lambda i: (0, 0)),
            pl.BlockSpec((DIM, DIM), lambda i: (0, 0)),
            pl.BlockSpec((1, DIM), lambda i: (0, 0)),
        ],
        out_specs=pl.BlockSpec((blk, DIM), lambda i: (i, 0)),
        out_shape=jax.ShapeDtypeStruct((AVP, DIM), jnp.float32),
    )(g, w1r, b1, w2, b2, w3, b3, w4p, b4p)


@jax.jit
def kernel(var_node_features, con_node_features, node_types, assoc_var, assoc_con,
           edge_index, edge_types, edge_features,
           vm_w1, vm_b1, vm_w2, vm_b2, cm_w1, cm_b1, cm_w2, cm_b2,
           conv1_wcons, conv1_root, conv1_bias,
           conv2_wcons, conv2_root, conv2_bias,
           conv3_wcons, conv3_root, conv3_bias,
           conv4_wcons, conv4_root, conv4_bias,
           fc1_w, fc1_b, fc2_w, fc2_b, fc3_w, fc3_b, fc4_w, fc4_b):
    f32 = jnp.float32

    io_mask = ((lax.broadcasted_iota(jnp.int32, (1, DIM), 1) == 0) |
               (lax.broadcasted_iota(jnp.int32, (1, DIM), 1) == DIM - 1)
               ).astype(f32)
    v2 = var_node_features * io_mask
    c2 = con_node_features * io_mask

    def pad_mlp_weights(w1, b1, w2, b2):
        w1p = jnp.pad(w1, ((0, 0), (0, 1)))
        b1p = jnp.pad(b1, (0, 1)).reshape(1, DIM)
        w2p = jnp.zeros((DIM, DIM), f32).at[: DIM - 1, : DIM - 1].set(w2)
        b2p = jnp.pad(b2, (0, 1)).reshape(1, DIM)
        return w1p, b1p, w2p, b2p

    n_emb = _embed(v2, *pad_mlp_weights(vm_w1, vm_b1, vm_w2, vm_b2))
    e_emb = _embed(c2, *pad_mlp_weights(cm_w1, cm_b1, cm_w2, cm_b2))
    table = jnp.concatenate([n_emb, e_emb, jnp.zeros((NP_ - N, DIM), f32)], axis=0)

    sel = jnp.full((N,), ZROW, jnp.int32)
    sel = sel.at[assoc_var].set(jnp.arange(N_VAR, dtype=jnp.int32))
    sel = sel.at[assoc_con].set(N_VAR + jnp.arange(N_CON, dtype=jnp.int32))
    sel_p = jnp.pad(sel, (0, NP_ - N), constant_values=ZROW).reshape(NW, 4, 80)
    x = _gather_rows(table, sel_p, 80, 4)

    perm = jnp.argsort(edge_index[1])
    src_s = jnp.take(edge_index[0], perm)
    dst_s = jnp.take(edge_index[1], perm)
    npad = NW * EPT - E
    pad_src = jnp.arange(npad, dtype=jnp.int32) % N
    pad_dst = N + jnp.arange(npad, dtype=jnp.int32) % (NP_ - N)
    src_p = jnp.concatenate([src_s, pad_src]).reshape(NW, NCHUNK, GW)
    dst_p = jnp.concatenate([dst_s, pad_dst]).reshape(NW, NCHUNK, GW)

    convs = ((conv1_wcons, conv1_root, conv1_bias),
             (conv2_wcons, conv2_root, conv2_bias),
             (conv3_wcons, conv3_root, conv3_bias),
             (conv4_wcons, conv4_root, conv4_bias))
    hs = [x]
    h = x
    for li, (wcons, root, bias) in enumerate(convs):
        proj = _msg_proj(h, wcons)
        part = _spmm(proj, src_p, dst_p)
        agg = part[0] + part[1]
        h = jax.nn.relu(agg + h @ root + bias)
        hs.append(h)

    av_p = jnp.pad(assoc_var, (0, AVP - N_VAR),
                   constant_values=ZROW).reshape(NW, 2, 96)
    g = _gather5(hs, av_p)
    w4p = jnp.pad(fc4_w, ((0, 0), (0, DIM - 1)))
    b4p = jnp.pad(fc4_b, (0, DIM - 1)).reshape(1, DIM)
    out = _fc_head(g, fc1_w.reshape(5, DIM, DIM), fc1_b.reshape(1, DIM),
                   fc2_w, fc2_b.reshape(1, DIM), fc3_w, fc3_b.reshape(1, DIM),
                   w4p, b4p)
    return out[:N_VAR, 0]


# speedup vs baseline: 15.5163x; 1.0665x over previous
import functools

import jax
import jax.numpy as jnp
from jax import lax
from jax.experimental import pallas as pl
from jax.experimental.pallas import tpu as pltpu
from jax.experimental.pallas import tpu_sc as plsc

DIM = 128
N = 10000
N_VAR = 6000
N_CON = 4000
E = 320000

NP_ = 10240
ZROW = N
EPT = NP_
NCHUNK = 128
GW = 80
NW = 32
STRIPE = NP_ // 16

AVP = 6144


def _mesh():
    return plsc.VectorSubcoreMesh(core_axis_name="c", subcore_axis_name="s")


def _spmm(h, src_p, dst_p):
    @functools.partial(
        pl.kernel,
        out_type=jax.ShapeDtypeStruct((2, NP_, DIM), jnp.float32),
        mesh=_mesh(),
        scratch_types=[
            pltpu.VMEM((NCHUNK // 2, GW), jnp.int32),
            pltpu.VMEM((NCHUNK // 2, GW), jnp.int32),
            pltpu.VMEM((GW, DIM), jnp.float32),
            pltpu.VMEM((GW, DIM), jnp.float32),
            pltpu.VMEM_SHARED((NP_, DIM), jnp.float32),
            pltpu.SemaphoreType.DMA,
            pltpu.SemaphoreType.DMA,
        ],
    )
    def k(h_hbm, src_hbm, dst_hbm, out_hbm, src_v, dst_v, bufa, bufb, acc,
          sema, semb):
        cid = lax.axis_index("c")
        sid = lax.axis_index("s")
        wid = cid * 16 + sid
        HC = NCHUNK // 2

        zv = jnp.zeros((16,), jnp.float32)

        def zrow(r, _):
            for j in range(DIM // 16):
                bufa[r, pl.ds(j * 16, 16)] = zv
            return 0

        lax.fori_loop(0, GW, zrow, 0)
        row0 = sid * STRIPE
        for j in range(STRIPE // GW):
            pltpu.sync_copy(bufa, acc.at[pl.ds(row0 + j * GW, GW)])
        plsc.subcore_barrier()

        for half in range(2):
            pltpu.sync_copy(src_hbm.at[wid, pl.ds(half * HC, HC)], src_v)
            pltpu.sync_copy(dst_hbm.at[wid, pl.ds(half * HC, HC)], dst_v)
            pltpu.make_async_copy(h_hbm.at[src_v.at[0]], bufa, sema).start()

            def pair(i, _):
                c = 2 * i
                pltpu.make_async_copy(h_hbm.at[src_v.at[c]], bufa, sema).wait()
                pltpu.make_async_copy(h_hbm.at[src_v.at[c + 1]], bufb,
                                      semb).start()
                pltpu.sync_copy(bufa, acc.at[dst_v.at[c]], add=True)
                pltpu.make_async_copy(h_hbm.at[src_v.at[c + 1]], bufb,
                                      semb).wait()

                @pl.when(c + 2 < HC)
                def _nx():
                    pltpu.make_async_copy(h_hbm.at[src_v.at[c + 2]], bufa,
                                          sema).start()

                pltpu.sync_copy(bufb, acc.at[dst_v.at[c + 1]], add=True)
                return 0

            lax.fori_loop(0, HC // 2, pair, 0)
        plsc.subcore_barrier()
        pltpu.sync_copy(acc.at[pl.ds(row0, STRIPE)],
                        out_hbm.at[cid, pl.ds(row0, STRIPE)])

    return k(h, src_p, dst_p)


def _gather_rows(table, sel_p, rows_per_chunk, chunks):
    rpt = rows_per_chunk * chunks

    @functools.partial(
        pl.kernel,
        out_type=jax.ShapeDtypeStruct((NW * rpt, DIM), jnp.float32),
        mesh=_mesh(),
        scratch_types=[
            pltpu.VMEM((chunks, rows_per_chunk), jnp.int32),
            pltpu.VMEM((chunks * rows_per_chunk, DIM), jnp.float32),
            pltpu.SemaphoreType.DMA,
        ],
    )
    def k(tbl_hbm, sel_hbm, out_hbm, idx_v, buf, sem):
        cid = lax.axis_index("c")
        sid = lax.axis_index("s")
        wid = cid * 16 + sid
        pltpu.sync_copy(sel_hbm.at[wid], idx_v)
        for c in range(chunks):
            pltpu.make_async_copy(
                tbl_hbm.at[idx_v.at[c]],
                buf.at[pl.ds(c * rows_per_chunk, rows_per_chunk)], sem).start()
        for c in range(chunks):
            pltpu.make_async_copy(
                tbl_hbm.at[idx_v.at[c]],
                buf.at[pl.ds(c * rows_per_chunk, rows_per_chunk)], sem).wait()
        pltpu.sync_copy(buf, out_hbm.at[pl.ds(wid * rpt, rpt)])

    return k(table, sel_p)


def _gather5(hs, av_p):
    @functools.partial(
        pl.kernel,
        out_type=jax.ShapeDtypeStruct((5, AVP, DIM), jnp.float32),
        mesh=_mesh(),
        scratch_types=[
            pltpu.VMEM((2, 96), jnp.int32),
            pltpu.VMEM((10 * 96, DIM), jnp.float32),
            pltpu.SemaphoreType.DMA,
            pltpu.SemaphoreType.DMA,
        ],
    )
    def k(h0, h1, h2, h3, h4, av_hbm, out_hbm, idx_v, buf, sg, sw):
        cid = lax.axis_index("c")
        sid = lax.axis_index("s")
        wid = cid * 16 + sid
        pltpu.sync_copy(av_hbm.at[wid], idx_v)
        tables = (h0, h1, h2, h3, h4)
        for t, hh in enumerate(tables):
            for c in range(2):
                pltpu.make_async_copy(
                    hh.at[idx_v.at[c]],
                    buf.at[pl.ds((2 * t + c) * 96, 96)], sg).start()
        for t, hh in enumerate(tables):
            for c in range(2):
                pltpu.make_async_copy(
                    hh.at[idx_v.at[c]],
                    buf.at[pl.ds((2 * t + c) * 96, 96)], sg).wait()
                pltpu.make_async_copy(
                    buf.at[pl.ds((2 * t + c) * 96, 96)],
                    out_hbm.at[t, pl.ds(wid * 192 + c * 96, 96)], sw).start()
        for t in range(5):
            for c in range(2):
                pltpu.make_async_copy(
                    buf.at[pl.ds((2 * t + c) * 96, 96)],
                    out_hbm.at[t, pl.ds(wid * 192 + c * 96, 96)], sw).wait()

    return k(*hs, av_p)


def _embed(v2, w1row, b1, w2, b2):
    rows = v2.shape[0]
    blk = 1000

    def body(v_ref, w1_ref, b1_ref, w2_ref, b2_ref, o_ref):
        v = v_ref[...]
        h1 = jax.nn.relu(v[:, 0:1] * w1_ref[...] + b1_ref[...])
        h2 = jnp.dot(h1, w2_ref[...], preferred_element_type=jnp.float32) + b2_ref[...]
        mask = lax.broadcasted_iota(jnp.int32, (1, DIM), 1) == (DIM - 1)
        o_ref[...] = jnp.where(mask, v, h2)

    return pl.pallas_call(
        body,
        grid=(rows // blk,),
        in_specs=[
            pl.BlockSpec((blk, DIM), lambda i: (i, 0)),
            pl.BlockSpec((1, DIM), lambda i: (0, 0)),
            pl.BlockSpec((1, DIM), lambda i: (0, 0)),
            pl.BlockSpec((DIM, DIM), lambda i: (0, 0)),
            pl.BlockSpec((1, DIM), lambda i: (0, 0)),
        ],
        out_specs=pl.BlockSpec((blk, DIM), lambda i: (i, 0)),
        out_shape=jax.ShapeDtypeStruct((rows, DIM), jnp.float32),
    )(v2, w1row, b1, w2, b2)


def _msg_proj(h, wcons):
    blk = 1280

    def body(h_ref, w_ref, o_ref):
        o_ref[...] = jnp.dot(h_ref[...], w_ref[...],
                             preferred_element_type=jnp.float32)

    return pl.pallas_call(
        body,
        grid=(NP_ // blk,),
        in_specs=[
            pl.BlockSpec((blk, DIM), lambda i: (i, 0)),
            pl.BlockSpec((DIM, DIM), lambda i: (0, 0)),
        ],
        out_specs=pl.BlockSpec((blk, DIM), lambda i: (i, 0)),
        out_shape=jax.ShapeDtypeStruct((NP_, DIM), jnp.float32),
    )(h, wcons)


def _conv_update(part, h, root, bias2d):
    blk = 1280

    def body(p_ref, h_ref, r_ref, b_ref, o_ref):
        a = p_ref[0] + p_ref[1]
        o_ref[...] = jax.nn.relu(
            a
            + jnp.dot(h_ref[...], r_ref[...], preferred_element_type=jnp.float32)
            + b_ref[...])

    return pl.pallas_call(
        body,
        grid=(NP_ // blk,),
        in_specs=[
            pl.BlockSpec((2, blk, DIM), lambda i: (0, i, 0)),
            pl.BlockSpec((blk, DIM), lambda i: (i, 0)),
            pl.BlockSpec((DIM, DIM), lambda i: (0, 0)),
            pl.BlockSpec((1, DIM), lambda i: (0, 0)),
        ],
        out_specs=pl.BlockSpec((blk, DIM), lambda i: (i, 0)),
        out_shape=jax.ShapeDtypeStruct((NP_, DIM), jnp.float32),
    )(part, h, root, bias2d)


def _fc_head(g, w1r, b1, w2, b2, w3, b3, w4p, b4p):
    blk = 768

    def body(g_ref, w1_ref, b1_ref, w2_ref, b2_ref, w3_ref, b3_ref,
             w4_ref, b4_ref, o_ref):
        acc = b1_ref[...] + jnp.zeros((blk, DIM), jnp.float32)
        for t in range(5):
            acc = acc + jnp.dot(g_ref[t], w1_ref[t],
                                preferred_element_type=jnp.float32)
        hh = jax.nn.relu(acc)
        hh = jax.nn.relu(jnp.dot(hh, w2_ref[...],
                                 preferred_element_type=jnp.float32) + b2_ref[...])
        hh = jax.nn.relu(jnp.dot(hh, w3_ref[...],
                                 preferred_element_type=jnp.float32) + b3_ref[...])
        o_ref[...] = jnp.dot(hh, w4_ref[...],
                             preferred_element_type=jnp.float32) + b4_ref[...]

    return pl.pallas_call(
        body,
        grid=(AVP // blk,),
        in_specs=[
            pl.BlockSpec((5, blk, DIM), lambda i: (0, i, 0)),
            pl.BlockSpec((5, DIM, DIM), lambda i: (0, 0, 0)),
            pl.BlockSpec((1, DIM), lambda i: (0, 0)),
            pl.BlockSpec((DIM, DIM), lambda i: (0, 0)),
            pl.BlockSpec((1, DIM), lambda i: (0, 0)),
            pl.BlockSpec((DIM, DIM), lambda i: (0, 0)),
            pl.BlockSpec((1, DIM), lambda i: (0, 0)),
            pl.BlockSpec((DIM, DIM), lambda i: (0, 0)),
            pl.BlockSpec((1, DIM), lambda i: (0, 0)),
        ],
        out_specs=pl.BlockSpec((blk, DIM), lambda i: (i, 0)),
        out_shape=jax.ShapeDtypeStruct((AVP, DIM), jnp.float32),
    )(g, w1r, b1, w2, b2, w3, b3, w4p, b4p)


@jax.jit
def kernel(var_node_features, con_node_features, node_types, assoc_var, assoc_con,
           edge_index, edge_types, edge_features,
           vm_w1, vm_b1, vm_w2, vm_b2, cm_w1, cm_b1, cm_w2, cm_b2,
           conv1_wcons, conv1_root, conv1_bias,
           conv2_wcons, conv2_root, conv2_bias,
           conv3_wcons, conv3_root, conv3_bias,
           conv4_wcons, conv4_root, conv4_bias,
           fc1_w, fc1_b, fc2_w, fc2_b, fc3_w, fc3_b, fc4_w, fc4_b):
    f32 = jnp.float32

    io_mask = ((lax.broadcasted_iota(jnp.int32, (1, DIM), 1) == 0) |
               (lax.broadcasted_iota(jnp.int32, (1, DIM), 1) == DIM - 1)
               ).astype(f32)
    v2 = var_node_features * io_mask
    c2 = con_node_features * io_mask

    def pad_mlp_weights(w1, b1, w2, b2):
        w1p = jnp.pad(w1, ((0, 0), (0, 1)))
        b1p = jnp.pad(b1, (0, 1)).reshape(1, DIM)
        w2p = jnp.zeros((DIM, DIM), f32).at[: DIM - 1, : DIM - 1].set(w2)
        b2p = jnp.pad(b2, (0, 1)).reshape(1, DIM)
        return w1p, b1p, w2p, b2p

    n_emb = _embed(v2, *pad_mlp_weights(vm_w1, vm_b1, vm_w2, vm_b2))
    e_emb = _embed(c2, *pad_mlp_weights(cm_w1, cm_b1, cm_w2, cm_b2))
    table = jnp.concatenate([n_emb, e_emb, jnp.zeros((NP_ - N, DIM), f32)], axis=0)

    sel = jnp.full((N,), ZROW, jnp.int32)
    sel = sel.at[assoc_var].set(jnp.arange(N_VAR, dtype=jnp.int32))
    sel = sel.at[assoc_con].set(N_VAR + jnp.arange(N_CON, dtype=jnp.int32))
    sel_p = jnp.pad(sel, (0, NP_ - N), constant_values=ZROW).reshape(NW, 4, 80)
    x = _gather_rows(table, sel_p, 80, 4)

    perm = jnp.argsort(edge_index[1])
    src_s = jnp.take(edge_index[0], perm)
    dst_s = jnp.take(edge_index[1], perm)
    npad = NW * EPT - E
    pad_src = jnp.arange(npad, dtype=jnp.int32) % N
    pad_dst = N + jnp.arange(npad, dtype=jnp.int32) % (NP_ - N)
    src_p = jnp.concatenate([src_s, pad_src]).reshape(NW, NCHUNK, GW)
    dst_p = jnp.concatenate([dst_s, pad_dst]).reshape(NW, NCHUNK, GW)

    convs = ((conv1_wcons, conv1_root, conv1_bias),
             (conv2_wcons, conv2_root, conv2_bias),
             (conv3_wcons, conv3_root, conv3_bias),
             (conv4_wcons, conv4_root, conv4_bias))
    hs = [x]
    h = x
    for li, (wcons, root, bias) in enumerate(convs):
        proj = _msg_proj(h, wcons)
        part = _spmm(proj, src_p, dst_p)
        agg = part[0] + part[1]
        h = jax.nn.relu(agg + h @ root + bias)
        hs.append(h)

    av_p = jnp.pad(assoc_var, (0, AVP - N_VAR),
                   constant_values=ZROW).reshape(NW, 2, 96)
    g = _gather5(hs, av_p)
    w4p = jnp.pad(fc4_w, ((0, 0), (0, DIM - 1)))
    b4p = jnp.pad(fc4_b, (0, DIM - 1)).reshape(1, DIM)
    out = _fc_head(g, fc1_w.reshape(5, DIM, DIM), fc1_b.reshape(1, DIM),
                   fc2_w, fc2_b.reshape(1, DIM), fc3_w, fc3_b.reshape(1, DIM),
                   w4p, b4p)
    return out[:N_VAR, 0]
